# Initial kernel scaffold; baseline (speedup 1.0000x reference)
import jax
import jax.numpy as jnp
from jax.experimental import pallas as pl


def kernel(logits, labels):
    raise NotImplementedError("write your pallas kernel here")


# speedup vs baseline: 30.5125x; 30.5125x over previous
import functools

import jax
import jax.numpy as jnp
from jax import lax
from jax.experimental import pallas as pl
from jax.experimental.pallas import tpu as pltpu
from jax.experimental.pallas import tpu_sc as plsc

N = 262144
C = 100
NB = 15
HB = 16
HTOT = C * HB

NW = 32
ROWS_W = N // NW
R = 128
NCHUNKS = ROWS_W // R
GROUPS = R // 16


def _sc_body(logits_hbm, labels_hbm, out_hbm, chunk_v, labels_v, rbuf,
             cnt_h, conf_h, acc_h):
    wid = lax.axis_index("s") * 2 + lax.axis_index("c")
    zero16 = jnp.zeros((16,), jnp.float32)
    ones16 = jnp.ones((16,), jnp.float32)
    lane = lax.broadcasted_iota(jnp.int32, (16,), 0)
    rowoff0 = lane * C

    def zero_body(i, _):
        cnt_h[pl.ds(i * 16, 16)] = zero16
        conf_h[pl.ds(i * 16, 16)] = zero16
        acc_h[pl.ds(i * 16, 16)] = zero16
        return 0
    lax.fori_loop(0, HTOT // 16, zero_body, 0)

    pltpu.sync_copy(labels_hbm.at[pl.ds(wid * ROWS_W, ROWS_W)], labels_v)

    def chunk_body(ci, _):
        row_base = wid * ROWS_W + ci * R
        pltpu.sync_copy(logits_hbm.at[pl.ds(row_base * C, R * C)], chunk_v)

        def group_a(g, _):
            ro = rowoff0 + g * (16 * C)

            def ja(j, s):
                return s + jnp.exp(plsc.load_gather(chunk_v, [ro + j]))

            s = lax.fori_loop(0, C, ja, zero16)
            rbuf[pl.ds(g * 16, 16)] = 1.0 / s
            return 0
        lax.fori_loop(0, GROUPS, group_a, 0)

        def group_b(g, _):
            ro = rowoff0 + g * (16 * C)
            r = rbuf[pl.ds(g * 16, 16)]

            def jb(j, _):
                e = jnp.exp(plsc.load_gather(chunk_v, [ro + j]))
                cv = e * r
                t = jnp.minimum((cv * float(NB)).astype(jnp.int32), NB - 1)
                seg = j * HB + t
                plsc.addupdate_scatter(cnt_h, [seg], ones16)
                plsc.addupdate_scatter(conf_h, [seg], cv)
                return 0
            lax.fori_loop(0, C, jb, 0)

            lbl = labels_v[pl.ds(ci * R + g * 16, 16)]
            e = jnp.exp(plsc.load_gather(chunk_v, [ro + lbl]))
            cv = e * r
            t = jnp.minimum((cv * float(NB)).astype(jnp.int32), NB - 1)
            plsc.addupdate_scatter(acc_h, [lbl * HB + t], ones16)
            return 0
        lax.fori_loop(0, GROUPS, group_b, 0)
        return 0
    lax.fori_loop(0, NCHUNKS, chunk_body, 0)

    pltpu.sync_copy(cnt_h, out_hbm.at[wid])
    pltpu.sync_copy(conf_h, out_hbm.at[NW + wid])
    pltpu.sync_copy(acc_h, out_hbm.at[2 * NW + wid])


@functools.partial(
    pl.kernel,
    out_type=jax.ShapeDtypeStruct((3 * NW, HTOT), jnp.float32),
    mesh=plsc.VectorSubcoreMesh(core_axis_name="c", subcore_axis_name="s"),
    scratch_types=[
        pltpu.VMEM((R * C,), jnp.float32),
        pltpu.VMEM((ROWS_W,), jnp.int32),
        pltpu.VMEM((R,), jnp.float32),
        pltpu.VMEM((HTOT,), jnp.float32),
        pltpu.VMEM((HTOT,), jnp.float32),
        pltpu.VMEM((HTOT,), jnp.float32),
    ],
    compiler_params=pltpu.CompilerParams(needs_layout_passes=False),
)
def _sc_hist(logits_hbm, labels_hbm, out_hbm, *scratch):
    _sc_body(logits_hbm, labels_hbm, out_hbm, *scratch)


def _finalize_body(h_ref, pc_ref, sce_ref):
    h = h_ref[...]
    counts = jnp.sum(h[0:NW], axis=0, keepdims=True)
    confs = jnp.sum(h[NW:2 * NW], axis=0, keepdims=True)
    accs = jnp.sum(h[2 * NW:3 * NW], axis=0, keepdims=True)
    safe = jnp.maximum(counts, 1.0)
    contrib = jnp.where(
        counts > 0.0,
        jnp.abs(confs / safe - accs / safe) * (counts * (1.0 / N)),
        0.0,
    )
    row = lax.broadcasted_iota(jnp.int32, (HTOT, C), 0)
    col = lax.broadcasted_iota(jnp.int32, (HTOT, C), 1)
    pick = (row // HB == col).astype(jnp.float32)
    pc = jnp.dot(contrib, pick, preferred_element_type=jnp.float32)
    pc_ref[...] = pc
    sce_ref[...] = jnp.sum(pc, axis=(0, 1), keepdims=True) * (1.0 / C)


def _finalize(hists):
    return pl.pallas_call(
        _finalize_body,
        out_shape=[
            jax.ShapeDtypeStruct((1, C), jnp.float32),
            jax.ShapeDtypeStruct((1, 1), jnp.float32),
        ],
    )(hists)


def kernel(logits, labels):
    hists = _sc_hist(logits.reshape(N * C), labels)
    pc, sce = _finalize(hists)
    return sce.reshape(()), pc.reshape(C)


# speedup vs baseline: 32.1209x; 1.0527x over previous
import functools

import jax
import jax.numpy as jnp
from jax import lax
from jax.experimental import pallas as pl
from jax.experimental.pallas import tpu as pltpu
from jax.experimental.pallas import tpu_sc as plsc

N = 262144
C = 100
NB = 15
HB = 16
HTOT = C * HB

NW = 32
ROWS_W = N // NW
R = 128
NCHUNKS = ROWS_W // R
GROUPS = R // 16
UA = 10
UB = 5


def _sc_body(logits_hbm, labels_hbm, out_hbm, chunk_v, labels_v, rbuf,
             cnt_h, conf_h, acc_h):
    wid = lax.axis_index("s") * 2 + lax.axis_index("c")
    zero16 = jnp.zeros((16,), jnp.float32)
    ones16 = jnp.ones((16,), jnp.float32)
    lane = lax.broadcasted_iota(jnp.int32, (16,), 0)
    rowoff0 = lane * C

    def zero_body(i, _):
        cnt_h[pl.ds(i * 16, 16)] = zero16
        conf_h[pl.ds(i * 16, 16)] = zero16
        acc_h[pl.ds(i * 16, 16)] = zero16
        return 0
    lax.fori_loop(0, HTOT // 16, zero_body, 0)

    pltpu.sync_copy(labels_hbm.at[pl.ds(wid * ROWS_W, ROWS_W)], labels_v)

    def chunk_body(ci, _):
        row_base = wid * ROWS_W + ci * R
        pltpu.sync_copy(logits_hbm.at[pl.ds(row_base * C, R * C)], chunk_v)

        def group_a(g, _):
            ro = rowoff0 + g * (16 * C)

            def ja(k, s):
                j = k * UA
                es = [jnp.exp(plsc.load_gather(chunk_v, [ro + (j + u)]))
                      for u in range(UA)]
                for e in es:
                    s = s + e
                return s

            s = lax.fori_loop(0, C // UA, ja, zero16)
            rbuf[pl.ds(g * 16, 16)] = 1.0 / s
            return 0
        lax.fori_loop(0, GROUPS, group_a, 0)

        def group_b(g, _):
            ro = rowoff0 + g * (16 * C)
            r = rbuf[pl.ds(g * 16, 16)]

            def jb(k, _):
                j = k * UB
                for u in range(UB):
                    e = jnp.exp(plsc.load_gather(chunk_v, [ro + (j + u)]))
                    cv = e * r
                    t = jnp.minimum((cv * float(NB)).astype(jnp.int32),
                                    NB - 1)
                    seg = (j + u) * HB + t
                    plsc.addupdate_scatter(cnt_h, [seg], ones16)
                    plsc.addupdate_scatter(conf_h, [seg], cv)
                return 0
            lax.fori_loop(0, C // UB, jb, 0)

            lbl = labels_v[pl.ds(ci * R + g * 16, 16)]
            e = jnp.exp(plsc.load_gather(chunk_v, [ro + lbl]))
            cv = e * r
            t = jnp.minimum((cv * float(NB)).astype(jnp.int32), NB - 1)
            plsc.addupdate_scatter(acc_h, [lbl * HB + t], ones16)
            return 0
        lax.fori_loop(0, GROUPS, group_b, 0)
        return 0
    lax.fori_loop(0, NCHUNKS, chunk_body, 0)

    pltpu.sync_copy(cnt_h, out_hbm.at[wid])
    pltpu.sync_copy(conf_h, out_hbm.at[NW + wid])
    pltpu.sync_copy(acc_h, out_hbm.at[2 * NW + wid])


@functools.partial(
    pl.kernel,
    out_type=jax.ShapeDtypeStruct((3 * NW, HTOT), jnp.float32),
    mesh=plsc.VectorSubcoreMesh(core_axis_name="c", subcore_axis_name="s"),
    scratch_types=[
        pltpu.VMEM((R * C,), jnp.float32),
        pltpu.VMEM((ROWS_W,), jnp.int32),
        pltpu.VMEM((R,), jnp.float32),
        pltpu.VMEM((HTOT,), jnp.float32),
        pltpu.VMEM((HTOT,), jnp.float32),
        pltpu.VMEM((HTOT,), jnp.float32),
    ],
    compiler_params=pltpu.CompilerParams(needs_layout_passes=False),
)
def _sc_hist(logits_hbm, labels_hbm, out_hbm, *scratch):
    _sc_body(logits_hbm, labels_hbm, out_hbm, *scratch)


def _finalize_body(h_ref, pc_ref, sce_ref):
    h = h_ref[...]
    counts = jnp.sum(h[0:NW], axis=0, keepdims=True)
    confs = jnp.sum(h[NW:2 * NW], axis=0, keepdims=True)
    accs = jnp.sum(h[2 * NW:3 * NW], axis=0, keepdims=True)
    safe = jnp.maximum(counts, 1.0)
    contrib = jnp.where(
        counts > 0.0,
        jnp.abs(confs / safe - accs / safe) * (counts * (1.0 / N)),
        0.0,
    )
    row = lax.broadcasted_iota(jnp.int32, (HTOT, C), 0)
    col = lax.broadcasted_iota(jnp.int32, (HTOT, C), 1)
    pick = (row // HB == col).astype(jnp.float32)
    pc = jnp.dot(contrib, pick, preferred_element_type=jnp.float32)
    pc_ref[...] = pc
    sce_ref[...] = jnp.sum(pc, axis=(0, 1), keepdims=True) * (1.0 / C)


def _finalize(hists):
    return pl.pallas_call(
        _finalize_body,
        out_shape=[
            jax.ShapeDtypeStruct((1, C), jnp.float32),
            jax.ShapeDtypeStruct((1, 1), jnp.float32),
        ],
    )(hists)


def kernel(logits, labels):
    hists = _sc_hist(logits.reshape(N * C), labels)
    pc, sce = _finalize(hists)
    return sce.reshape(()), pc.reshape(C)
